# baseline (speedup 1.0000x reference)
import functools
import math

import jax
import jax.numpy as jnp
import numpy as np
from jax.experimental import pallas as pl
from jax.experimental.pallas import tpu as pltpu

D = 512
H = 8
L = 6
DFF = 2048
V = 32000
B = 2
S = 1024
DH = D // H
PAD_ID = 0
EMB_SCALE = math.sqrt(D)
NEG = -1e9

QC = 256
NCH = S // QC
BT = 256
NT = 3200
FT = 512


def _posenc(s, d):
    pos = np.arange(s)[:, None].astype(np.float32)
    i = np.arange(0, d, 2)[None, :].astype(np.float32)
    ang = pos / (10000.0 ** (i / d))
    pe = np.zeros((s, d), np.float32)
    pe[:, 0::2] = np.sin(ang)
    pe[:, 1::2] = np.cos(ang)
    return pe


_PE = _posenc(S, D)


def _ln(y, s, b):
    mu = jnp.mean(y, axis=-1, keepdims=True)
    d = y - mu
    var = jnp.mean(d * d, axis=-1, keepdims=True)
    return d * jax.lax.rsqrt(var + 1e-5) * s + b


def _dot(a, w):
    return jnp.dot(a, w, preferred_element_type=jnp.float32)


def _pad_cols(w, nsplit):
    parts = []
    for i in range(nsplit):
        blk = w[:, i * D:(i + 1) * D].reshape(w.shape[0], H, DH)
        blk = jnp.pad(blk, ((0, 0), (0, 0), (0, DHP - DH)))
        parts.append(blk.reshape(w.shape[0], H * DHP))
    return jnp.concatenate(parts, axis=1)


def _pad_rows(w):
    blk = w.reshape(H, DH, D)
    blk = jnp.pad(blk, ((0, 0), (0, DHP - DH), (0, 0)))
    return blk.reshape(H * DHP, D)


def _embed_body(ids_ref, emb_hbm, pe_ref, out_ref, buf, sem):
    i = pl.program_id(0)
    base = i * BT
    copies = []
    for t in range(BT):
        idx = ids_ref[base + t]
        cp = pltpu.make_async_copy(emb_hbm.at[idx], buf.at[t], sem)
        cp.start()
        copies.append(cp)
    for cp in copies:
        cp.wait()
    out_ref[...] = buf[...] * EMB_SCALE + pe_ref[...]


def _embed(ids_flat, emb):
    n = ids_flat.shape[0]
    pe_blocks = S // BT
    return pl.pallas_call(
        _embed_body,
        out_shape=jax.ShapeDtypeStruct((n, D), jnp.float32),
        grid_spec=pltpu.PrefetchScalarGridSpec(
            num_scalar_prefetch=1,
            grid=(n // BT,),
            in_specs=[
                pl.BlockSpec(memory_space=pl.ANY),
                pl.BlockSpec((BT, D), lambda i, ids: (i % pe_blocks, 0)),
            ],
            out_specs=pl.BlockSpec((BT, D), lambda i, ids: (i, 0)),
            scratch_shapes=[
                pltpu.VMEM((BT, D), jnp.float32),
                pltpu.SemaphoreType.DMA,
            ],
        ),
        compiler_params=pltpu.CompilerParams(
            dimension_semantics=("arbitrary",),
        ),
        name="embed_gather",
    )(ids_flat, emb, jnp.asarray(_PE))


DHP = 128


def _attn_math(q_ref, q_off, kv_ref, k_off, v_off, padf, causal, o_scr):
    scale = DH ** -0.5
    for h in range(H):
        k = kv_ref[:, k_off + h * DHP:k_off + (h + 1) * DHP]
        v = kv_ref[:, v_off + h * DHP:v_off + (h + 1) * DHP]
        for r0 in range(0, S, QC):
            q = q_ref[r0:r0 + QC, q_off + h * DHP:q_off + (h + 1) * DHP]
            sc = jax.lax.dot_general(q, k, (((1,), (1,)), ((), ())),
                                     preferred_element_type=jnp.float32)
            sc = sc * scale
            if causal:
                rows = jax.lax.broadcasted_iota(jnp.int32, (QC, S), 0) + r0
                cols = jax.lax.broadcasted_iota(jnp.int32, (QC, S), 1)
                sc = jnp.where(cols > rows, NEG, sc)
            else:
                sc = jnp.where(padf > 0.5, NEG, sc)
            m = jnp.max(sc, axis=-1, keepdims=True)
            p = jnp.exp(sc - m)
            l = jnp.sum(p, axis=-1, keepdims=True)
            p = p / l
            o_scr[r0:r0 + QC, h * DHP:(h + 1) * DHP] = _dot(p, v)


def _self_attn_body(x_ref, wqkv_ref, wo_ref, lns_ref, lnb_ref, padf_ref,
                    out_ref, qkv_scr, o_scr, *, causal):
    qkv_scr[...] = _dot(x_ref[0], wqkv_ref[...])
    hp = H * DHP
    _attn_math(qkv_scr, 0, qkv_scr, hp, 2 * hp, padf_ref[0], causal, o_scr)
    s = lns_ref[...]
    b = lnb_ref[...]
    for r0 in range(0, S, QC):
        proj = _dot(o_scr[r0:r0 + QC, :], wo_ref[...])
        y = x_ref[0, r0:r0 + QC, :] + proj
        out_ref[0, r0:r0 + QC, :] = _ln(y, s, b)


def _self_attn(x, wqkv, wo, lns, lnb, padf, causal):
    return pl.pallas_call(
        functools.partial(_self_attn_body, causal=causal),
        out_shape=jax.ShapeDtypeStruct((B, S, D), jnp.float32),
        grid=(B,),
        in_specs=[
            pl.BlockSpec((1, S, D), lambda b: (b, 0, 0)),
            pl.BlockSpec((D, 3 * H * DHP), lambda b: (0, 0)),
            pl.BlockSpec((H * DHP, D), lambda b: (0, 0)),
            pl.BlockSpec((1, D), lambda b: (0, 0)),
            pl.BlockSpec((1, D), lambda b: (0, 0)),
            pl.BlockSpec((1, 1, S), lambda b: (b, 0, 0)),
        ],
        out_specs=pl.BlockSpec((1, S, D), lambda b: (b, 0, 0)),
        scratch_shapes=[
            pltpu.VMEM((S, 3 * H * DHP), jnp.float32),
            pltpu.VMEM((S, H * DHP), jnp.float32),
        ],
        compiler_params=pltpu.CompilerParams(
            dimension_semantics=("parallel",),
            vmem_limit_bytes=48 * 1024 * 1024,
        ),
        name="self_attn_causal" if causal else "self_attn_pad",
    )(x, wqkv, wo, lns, lnb, padf)


def _cross_attn_body(y_ref, enc_ref, wq_ref, wkv_ref, woc_ref, lns_ref,
                     lnb_ref, padf_ref, out_ref, q_scr, kv_scr, o_scr):
    q_scr[...] = _dot(y_ref[0], wq_ref[...])
    kv_scr[...] = _dot(enc_ref[0], wkv_ref[...])
    _attn_math(q_scr, 0, kv_scr, 0, H * DHP, padf_ref[0], False, o_scr)
    s = lns_ref[...]
    b = lnb_ref[...]
    for r0 in range(0, S, QC):
        proj = _dot(o_scr[r0:r0 + QC, :], woc_ref[...])
        y = y_ref[0, r0:r0 + QC, :] + proj
        out_ref[0, r0:r0 + QC, :] = _ln(y, s, b)


def _cross_attn(y, enc_out, wq, wkv, woc, lns, lnb, padf):
    return pl.pallas_call(
        _cross_attn_body,
        out_shape=jax.ShapeDtypeStruct((B, S, D), jnp.float32),
        grid=(B,),
        in_specs=[
            pl.BlockSpec((1, S, D), lambda b: (b, 0, 0)),
            pl.BlockSpec((1, S, D), lambda b: (b, 0, 0)),
            pl.BlockSpec((D, H * DHP), lambda b: (0, 0)),
            pl.BlockSpec((D, 2 * H * DHP), lambda b: (0, 0)),
            pl.BlockSpec((H * DHP, D), lambda b: (0, 0)),
            pl.BlockSpec((1, D), lambda b: (0, 0)),
            pl.BlockSpec((1, D), lambda b: (0, 0)),
            pl.BlockSpec((1, 1, S), lambda b: (b, 0, 0)),
        ],
        out_specs=pl.BlockSpec((1, S, D), lambda b: (b, 0, 0)),
        scratch_shapes=[
            pltpu.VMEM((S, H * DHP), jnp.float32),
            pltpu.VMEM((S, 2 * H * DHP), jnp.float32),
            pltpu.VMEM((S, H * DHP), jnp.float32),
        ],
        compiler_params=pltpu.CompilerParams(
            dimension_semantics=("parallel",),
            vmem_limit_bytes=48 * 1024 * 1024,
        ),
        name="cross_attn",
    )(y, enc_out, wq, wkv, woc, lns, lnb, padf)


def _ffn_body(x_ref, w1_ref, b1_ref, w2_ref, b2_ref, lns_ref, lnb_ref,
              out_ref, h_scr):
    h_scr[...] = jnp.maximum(
        _dot(x_ref[...], w1_ref[...]) + b1_ref[...], 0.0)
    y = _dot(h_scr[...], w2_ref[...]) + b2_ref[...] + x_ref[...]
    out_ref[...] = _ln(y, lns_ref[...], lnb_ref[...])


def _ffn(x2d, w1, b1, w2, b2, lns, lnb):
    n = x2d.shape[0]
    return pl.pallas_call(
        _ffn_body,
        out_shape=jax.ShapeDtypeStruct((n, D), jnp.float32),
        grid=(n // FT,),
        in_specs=[
            pl.BlockSpec((FT, D), lambda i: (i, 0)),
            pl.BlockSpec((D, DFF), lambda i: (0, 0)),
            pl.BlockSpec((1, DFF), lambda i: (0, 0)),
            pl.BlockSpec((DFF, D), lambda i: (0, 0)),
            pl.BlockSpec((1, D), lambda i: (0, 0)),
            pl.BlockSpec((1, D), lambda i: (0, 0)),
            pl.BlockSpec((1, D), lambda i: (0, 0)),
        ],
        out_specs=pl.BlockSpec((FT, D), lambda i: (i, 0)),
        scratch_shapes=[pltpu.VMEM((FT, DFF), jnp.float32)],
        compiler_params=pltpu.CompilerParams(
            dimension_semantics=("parallel",),
            vmem_limit_bytes=48 * 1024 * 1024,
        ),
        name="ffn_block",
    )(x2d, w1, b1, w2, b2, lns, lnb)


def _logits_body(x_ref, w_ref, b_ref, out_ref):
    out_ref[...] = (jnp.dot(x_ref[...], w_ref[...],
                            preferred_element_type=jnp.float32)
                    + b_ref[...])


def _logits(x2d, fc_w, fc_b):
    n = x2d.shape[0]
    mt = n // 2
    return pl.pallas_call(
        _logits_body,
        out_shape=jax.ShapeDtypeStruct((n, V), jnp.float32),
        grid=(V // NT, 2),
        in_specs=[
            pl.BlockSpec((mt, D), lambda j, i: (i, 0)),
            pl.BlockSpec((D, NT), lambda j, i: (0, j)),
            pl.BlockSpec((1, NT), lambda j, i: (0, j)),
        ],
        out_specs=pl.BlockSpec((mt, NT), lambda j, i: (i, j)),
        compiler_params=pltpu.CompilerParams(
            dimension_semantics=("parallel", "arbitrary"),
            vmem_limit_bytes=56 * 1024 * 1024,
        ),
        name="vocab_proj",
    )(x2d, fc_w, fc_b.reshape(1, V))


def kernel(batch_src, trg_teacher, src_emb, trg_emb, fc_w, fc_b,
           enc_wqkv, enc_wo, enc_ln1s, enc_ln1b, enc_w1, enc_b1, enc_w2,
           enc_b2, enc_ln2s, enc_ln2b,
           dec_wqkv, dec_wo, dec_ln1s, dec_ln1b, dec_wq, dec_wkv, dec_woc,
           dec_ln2s, dec_ln2b, dec_w1, dec_b1, dec_w2, dec_b2, dec_ln3s,
           dec_ln3b):
    padf = (batch_src == PAD_ID).astype(jnp.float32).reshape(B, 1, S)
    zeros_pad = jnp.zeros_like(padf)

    x = _embed(batch_src.reshape(-1), src_emb).reshape(B, S, D)
    for l in range(L):
        x = _self_attn(x, _pad_cols(enc_wqkv[l], 3), _pad_rows(enc_wo[l]),
                       enc_ln1s[l].reshape(1, D), enc_ln1b[l].reshape(1, D),
                       padf, causal=False)
        x = _ffn(x.reshape(B * S, D), enc_w1[l], enc_b1[l].reshape(1, DFF),
                 enc_w2[l], enc_b2[l].reshape(1, D),
                 enc_ln2s[l].reshape(1, D),
                 enc_ln2b[l].reshape(1, D)).reshape(B, S, D)
    enc_out = x

    y = _embed(trg_teacher.reshape(-1), trg_emb).reshape(B, S, D)
    for l in range(L):
        y = _self_attn(y, _pad_cols(dec_wqkv[l], 3), _pad_rows(dec_wo[l]),
                       dec_ln1s[l].reshape(1, D), dec_ln1b[l].reshape(1, D),
                       zeros_pad, causal=True)
        y = _cross_attn(y, enc_out, _pad_cols(dec_wq[l], 1),
                        _pad_cols(dec_wkv[l], 2), _pad_rows(dec_woc[l]),
                        dec_ln2s[l].reshape(1, D), dec_ln2b[l].reshape(1, D),
                        padf)
        y = _ffn(y.reshape(B * S, D), dec_w1[l], dec_b1[l].reshape(1, DFF),
                 dec_w2[l], dec_b2[l].reshape(1, D),
                 dec_ln3s[l].reshape(1, D),
                 dec_ln3b[l].reshape(1, D)).reshape(B, S, D)

    return _logits(y.reshape(B * S, D), fc_w, fc_b).reshape(B, S, V)


# speedup vs baseline: 1.3002x; 1.3002x over previous
import functools
import math

import jax
import jax.numpy as jnp
import numpy as np
from jax.experimental import pallas as pl
from jax.experimental.pallas import tpu as pltpu

D = 512
H = 8
L = 6
DFF = 2048
V = 32000
B = 2
S = 1024
DH = D // H
PAD_ID = 0
EMB_SCALE = math.sqrt(D)
NEG = -1e9

QC = 256
NCH = S // QC
BT = 256
NT = 3200
FT = 512


def _posenc(s, d):
    pos = np.arange(s)[:, None].astype(np.float32)
    i = np.arange(0, d, 2)[None, :].astype(np.float32)
    ang = pos / (10000.0 ** (i / d))
    pe = np.zeros((s, d), np.float32)
    pe[:, 0::2] = np.sin(ang)
    pe[:, 1::2] = np.cos(ang)
    return pe


_PE = _posenc(S, D)


def _ln(y, s, b):
    mu = jnp.mean(y, axis=-1, keepdims=True)
    d = y - mu
    var = jnp.mean(d * d, axis=-1, keepdims=True)
    return d * jax.lax.rsqrt(var + 1e-5) * s + b


def _dot(a, w):
    return jnp.dot(a, w, preferred_element_type=jnp.float32)


def _pad_cols(w, nsplit):
    parts = []
    for i in range(nsplit):
        blk = w[:, i * D:(i + 1) * D].reshape(w.shape[0], H, DH)
        blk = jnp.pad(blk, ((0, 0), (0, 0), (0, DHP - DH)))
        parts.append(blk.reshape(w.shape[0], H * DHP))
    return jnp.concatenate(parts, axis=1)


def _pad_rows(w):
    blk = w.reshape(H, DH, D)
    blk = jnp.pad(blk, ((0, 0), (0, DHP - DH), (0, 0)))
    return blk.reshape(H * DHP, D)


def _embed_body(ids_ref, emb_hbm, pe_ref, out_ref, buf, sem):
    i = pl.program_id(0)
    base = i * BT
    copies = []
    for t in range(BT):
        idx = ids_ref[base + t]
        cp = pltpu.make_async_copy(emb_hbm.at[idx], buf.at[t], sem)
        cp.start()
        copies.append(cp)
    for cp in copies:
        cp.wait()
    out_ref[...] = buf[...] * EMB_SCALE + pe_ref[...]


def _embed(ids_flat, emb):
    n = ids_flat.shape[0]
    pe_blocks = S // BT
    return pl.pallas_call(
        _embed_body,
        out_shape=jax.ShapeDtypeStruct((n, D), jnp.float32),
        grid_spec=pltpu.PrefetchScalarGridSpec(
            num_scalar_prefetch=1,
            grid=(n // BT,),
            in_specs=[
                pl.BlockSpec(memory_space=pl.ANY),
                pl.BlockSpec((BT, D), lambda i, ids: (i % pe_blocks, 0)),
            ],
            out_specs=pl.BlockSpec((BT, D), lambda i, ids: (i, 0)),
            scratch_shapes=[
                pltpu.VMEM((BT, D), jnp.float32),
                pltpu.SemaphoreType.DMA,
            ],
        ),
        compiler_params=pltpu.CompilerParams(
            dimension_semantics=("arbitrary",),
        ),
        name="embed_gather",
    )(ids_flat, emb, jnp.asarray(_PE))


def _attn_math(q_ref, q_off, kv_ref, k_off, v_off, padf, causal, o_scr):
    scale = DH ** -0.5
    for h in range(H):
        k = kv_ref[:, k_off + h * DH:k_off + (h + 1) * DH]
        v = kv_ref[:, v_off + h * DH:v_off + (h + 1) * DH]
        for r0 in range(0, S, QC):
            q = q_ref[r0:r0 + QC, q_off + h * DH:q_off + (h + 1) * DH]
            sc = jax.lax.dot_general(q, k, (((1,), (1,)), ((), ())),
                                     preferred_element_type=jnp.float32)
            sc = sc * scale
            if causal:
                rows = jax.lax.broadcasted_iota(jnp.int32, (QC, S), 0) + r0
                cols = jax.lax.broadcasted_iota(jnp.int32, (QC, S), 1)
                sc = jnp.where(cols > rows, NEG, sc)
            else:
                sc = jnp.where(padf > 0.5, NEG, sc)
            m = jnp.max(sc, axis=-1, keepdims=True)
            p = jnp.exp(sc - m)
            l = jnp.sum(p, axis=-1, keepdims=True)
            p = p / l
            o_scr[r0:r0 + QC, h * DH:(h + 1) * DH] = _dot(p, v)


def _ffn_tail(state_scr, w1_ref, fb1, w2_ref, fb2, lns, lnb, out_ref, h_scr):
    ft = h_scr.shape[0]
    for r0 in range(0, S, ft):
        xc = state_scr[r0:r0 + ft, :]
        h_scr[...] = jnp.maximum(_dot(xc, w1_ref[...]) + fb1, 0.0)
        y = _dot(h_scr[...], w2_ref[...]) + fb2 + xc
        out_ref[0, r0:r0 + ft, :] = _ln(y, lns, lnb)


def _self_attn_body(x_ref, wqkv_ref, wo_ref, lns_ref, lnb_ref, padf_ref,
                    w1_ref, fb1_ref, w2_ref, fb2_ref, l2s_ref, l2b_ref,
                    out_ref, qkv_scr, o_scr, h_scr, *, causal, with_ffn):
    qkv_scr[...] = _dot(x_ref[0], wqkv_ref[...])
    _attn_math(qkv_scr, 0, qkv_scr, D, 2 * D, padf_ref[0], causal, o_scr)
    s = lns_ref[...]
    b = lnb_ref[...]
    for r0 in range(0, S, QC):
        proj = _dot(o_scr[r0:r0 + QC, :], wo_ref[...])
        y = x_ref[0, r0:r0 + QC, :] + proj
        x2 = _ln(y, s, b)
        if with_ffn:
            o_scr[r0:r0 + QC, :] = x2
        else:
            out_ref[0, r0:r0 + QC, :] = x2
    if with_ffn:
        _ffn_tail(o_scr, w1_ref, fb1_ref[...], w2_ref, fb2_ref[...],
                  l2s_ref[...], l2b_ref[...], out_ref, h_scr)


def _self_attn(x, wqkv, wo, lns, lnb, padf, causal,
               w1=None, fb1=None, w2=None, fb2=None, l2s=None, l2b=None):
    with_ffn = w1 is not None
    if not with_ffn:
        w1 = jnp.zeros((8, DFF), jnp.float32)
        fb1 = jnp.zeros((1, DFF), jnp.float32)
        w2 = jnp.zeros((8, D), jnp.float32)
        fb2 = jnp.zeros((1, D), jnp.float32)
        l2s = jnp.zeros((1, D), jnp.float32)
        l2b = jnp.zeros((1, D), jnp.float32)
    kd = w1.shape[0]
    return pl.pallas_call(
        functools.partial(_self_attn_body, causal=causal, with_ffn=with_ffn),
        out_shape=jax.ShapeDtypeStruct((B, S, D), jnp.float32),
        grid=(B,),
        in_specs=[
            pl.BlockSpec((1, S, D), lambda b: (b, 0, 0)),
            pl.BlockSpec((D, 3 * D), lambda b: (0, 0)),
            pl.BlockSpec((D, D), lambda b: (0, 0)),
            pl.BlockSpec((1, D), lambda b: (0, 0)),
            pl.BlockSpec((1, D), lambda b: (0, 0)),
            pl.BlockSpec((1, 1, S), lambda b: (b, 0, 0)),
            pl.BlockSpec((kd, DFF), lambda b: (0, 0)),
            pl.BlockSpec((1, DFF), lambda b: (0, 0)),
            pl.BlockSpec((w2.shape[0], D), lambda b: (0, 0)),
            pl.BlockSpec((1, D), lambda b: (0, 0)),
            pl.BlockSpec((1, D), lambda b: (0, 0)),
            pl.BlockSpec((1, D), lambda b: (0, 0)),
        ],
        out_specs=pl.BlockSpec((1, S, D), lambda b: (b, 0, 0)),
        scratch_shapes=[
            pltpu.VMEM((S, 3 * D), jnp.float32),
            pltpu.VMEM((S, D), jnp.float32),
            pltpu.VMEM((FT, DFF), jnp.float32),
        ],
        compiler_params=pltpu.CompilerParams(
            dimension_semantics=("parallel",),
            vmem_limit_bytes=52 * 1024 * 1024,
        ),
        name=("enc_layer" if with_ffn else "self_attn_causal"),
    )(x, wqkv, wo, lns, lnb, padf, w1, fb1, w2, fb2, l2s, l2b)


def _cross_attn_body(y_ref, enc_ref, wq_ref, wkv_ref, woc_ref, lns_ref,
                     lnb_ref, padf_ref, w1_ref, fb1_ref, w2_ref, fb2_ref,
                     l2s_ref, l2b_ref, out_ref, q_scr, kv_scr, o_scr, h_scr):
    q_scr[...] = _dot(y_ref[0], wq_ref[...])
    kv_scr[...] = _dot(enc_ref[0], wkv_ref[...])
    _attn_math(q_scr, 0, kv_scr, 0, D, padf_ref[0], False, o_scr)
    s = lns_ref[...]
    b = lnb_ref[...]
    for r0 in range(0, S, QC):
        proj = _dot(o_scr[r0:r0 + QC, :], woc_ref[...])
        y = y_ref[0, r0:r0 + QC, :] + proj
        o_scr[r0:r0 + QC, :] = _ln(y, s, b)
    _ffn_tail(o_scr, w1_ref, fb1_ref[...], w2_ref, fb2_ref[...],
              l2s_ref[...], l2b_ref[...], out_ref, h_scr)


def _cross_attn(y, enc_out, wq, wkv, woc, lns, lnb, padf,
                w1, fb1, w2, fb2, l2s, l2b):
    return pl.pallas_call(
        _cross_attn_body,
        out_shape=jax.ShapeDtypeStruct((B, S, D), jnp.float32),
        grid=(B,),
        in_specs=[
            pl.BlockSpec((1, S, D), lambda b: (b, 0, 0)),
            pl.BlockSpec((1, S, D), lambda b: (b, 0, 0)),
            pl.BlockSpec((D, D), lambda b: (0, 0)),
            pl.BlockSpec((D, 2 * D), lambda b: (0, 0)),
            pl.BlockSpec((D, D), lambda b: (0, 0)),
            pl.BlockSpec((1, D), lambda b: (0, 0)),
            pl.BlockSpec((1, D), lambda b: (0, 0)),
            pl.BlockSpec((1, 1, S), lambda b: (b, 0, 0)),
            pl.BlockSpec((D, DFF), lambda b: (0, 0)),
            pl.BlockSpec((1, DFF), lambda b: (0, 0)),
            pl.BlockSpec((DFF, D), lambda b: (0, 0)),
            pl.BlockSpec((1, D), lambda b: (0, 0)),
            pl.BlockSpec((1, D), lambda b: (0, 0)),
            pl.BlockSpec((1, D), lambda b: (0, 0)),
        ],
        out_specs=pl.BlockSpec((1, S, D), lambda b: (b, 0, 0)),
        scratch_shapes=[
            pltpu.VMEM((S, D), jnp.float32),
            pltpu.VMEM((S, 2 * D), jnp.float32),
            pltpu.VMEM((S, D), jnp.float32),
            pltpu.VMEM((FT, DFF), jnp.float32),
        ],
        compiler_params=pltpu.CompilerParams(
            dimension_semantics=("parallel",),
            vmem_limit_bytes=52 * 1024 * 1024,
        ),
        name="dec_cross_ffn",
    )(y, enc_out, wq, wkv, woc, lns, lnb, padf, w1, fb1, w2, fb2, l2s, l2b)


def _ffn_body(x_ref, w1_ref, b1_ref, w2_ref, b2_ref, lns_ref, lnb_ref,
              out_ref, h_scr):
    h_scr[...] = jnp.maximum(
        _dot(x_ref[...], w1_ref[...]) + b1_ref[...], 0.0)
    y = _dot(h_scr[...], w2_ref[...]) + b2_ref[...] + x_ref[...]
    out_ref[...] = _ln(y, lns_ref[...], lnb_ref[...])


def _ffn(x2d, w1, b1, w2, b2, lns, lnb):
    n = x2d.shape[0]
    return pl.pallas_call(
        _ffn_body,
        out_shape=jax.ShapeDtypeStruct((n, D), jnp.float32),
        grid=(n // FT,),
        in_specs=[
            pl.BlockSpec((FT, D), lambda i: (i, 0)),
            pl.BlockSpec((D, DFF), lambda i: (0, 0)),
            pl.BlockSpec((1, DFF), lambda i: (0, 0)),
            pl.BlockSpec((DFF, D), lambda i: (0, 0)),
            pl.BlockSpec((1, D), lambda i: (0, 0)),
            pl.BlockSpec((1, D), lambda i: (0, 0)),
            pl.BlockSpec((1, D), lambda i: (0, 0)),
        ],
        out_specs=pl.BlockSpec((FT, D), lambda i: (i, 0)),
        scratch_shapes=[pltpu.VMEM((FT, DFF), jnp.float32)],
        compiler_params=pltpu.CompilerParams(
            dimension_semantics=("parallel",),
            vmem_limit_bytes=48 * 1024 * 1024,
        ),
        name="ffn_block",
    )(x2d, w1, b1, w2, b2, lns, lnb)


def _logits_body(x_ref, w_ref, b_ref, out_ref):
    out_ref[...] = (jnp.dot(x_ref[...], w_ref[...],
                            preferred_element_type=jnp.float32)
                    + b_ref[...])


def _logits(x2d, fc_w, fc_b):
    n = x2d.shape[0]
    mt = n // 2
    return pl.pallas_call(
        _logits_body,
        out_shape=jax.ShapeDtypeStruct((n, V), jnp.float32),
        grid=(V // NT, 2),
        in_specs=[
            pl.BlockSpec((mt, D), lambda j, i: (i, 0)),
            pl.BlockSpec((D, NT), lambda j, i: (0, j)),
            pl.BlockSpec((1, NT), lambda j, i: (0, j)),
        ],
        out_specs=pl.BlockSpec((mt, NT), lambda j, i: (i, j)),
        compiler_params=pltpu.CompilerParams(
            dimension_semantics=("parallel", "arbitrary"),
            vmem_limit_bytes=56 * 1024 * 1024,
        ),
        name="vocab_proj",
    )(x2d, fc_w, fc_b.reshape(1, V))


def kernel(batch_src, trg_teacher, src_emb, trg_emb, fc_w, fc_b,
           enc_wqkv, enc_wo, enc_ln1s, enc_ln1b, enc_w1, enc_b1, enc_w2,
           enc_b2, enc_ln2s, enc_ln2b,
           dec_wqkv, dec_wo, dec_ln1s, dec_ln1b, dec_wq, dec_wkv, dec_woc,
           dec_ln2s, dec_ln2b, dec_w1, dec_b1, dec_w2, dec_b2, dec_ln3s,
           dec_ln3b):
    padf = (batch_src == PAD_ID).astype(jnp.float32).reshape(B, 1, S)
    zeros_pad = jnp.zeros_like(padf)

    x = _embed(batch_src.reshape(-1), src_emb).reshape(B, S, D)
    for l in range(L):
        x = _self_attn(x, enc_wqkv[l], enc_wo[l],
                       enc_ln1s[l].reshape(1, D), enc_ln1b[l].reshape(1, D),
                       padf, False,
                       enc_w1[l], enc_b1[l].reshape(1, DFF), enc_w2[l],
                       enc_b2[l].reshape(1, D), enc_ln2s[l].reshape(1, D),
                       enc_ln2b[l].reshape(1, D))
    enc_out = x

    y = _embed(trg_teacher.reshape(-1), trg_emb).reshape(B, S, D)
    for l in range(L):
        y = _self_attn(y, dec_wqkv[l], dec_wo[l],
                       dec_ln1s[l].reshape(1, D), dec_ln1b[l].reshape(1, D),
                       zeros_pad, causal=True)
        y = _cross_attn(y, enc_out, dec_wq[l], dec_wkv[l], dec_woc[l],
                        dec_ln2s[l].reshape(1, D), dec_ln2b[l].reshape(1, D),
                        padf,
                        dec_w1[l], dec_b1[l].reshape(1, DFF), dec_w2[l],
                        dec_b2[l].reshape(1, D), dec_ln3s[l].reshape(1, D),
                        dec_ln3b[l].reshape(1, D))

    return _logits(y.reshape(B * S, D), fc_w, fc_b).reshape(B, S, V)


# speedup vs baseline: 1.3498x; 1.0382x over previous
import functools
import math

import jax
import jax.numpy as jnp
import numpy as np
from jax.experimental import pallas as pl
from jax.experimental.pallas import tpu as pltpu

D = 512
H = 8
L = 6
DFF = 2048
V = 32000
B = 2
S = 1024
DH = D // H
PAD_ID = 0
EMB_SCALE = math.sqrt(D)
NEG = -1e9

QC = 256
NCH = S // QC
BT = 256
NT = 3200
FT = 512


def _posenc(s, d):
    pos = np.arange(s)[:, None].astype(np.float32)
    i = np.arange(0, d, 2)[None, :].astype(np.float32)
    ang = pos / (10000.0 ** (i / d))
    pe = np.zeros((s, d), np.float32)
    pe[:, 0::2] = np.sin(ang)
    pe[:, 1::2] = np.cos(ang)
    return pe


_PE = _posenc(S, D)


def _ln(y, s, b):
    mu = jnp.mean(y, axis=-1, keepdims=True)
    d = y - mu
    var = jnp.mean(d * d, axis=-1, keepdims=True)
    return d * jax.lax.rsqrt(var + 1e-5) * s + b


def _dot(a, w):
    return jnp.dot(a, w, preferred_element_type=jnp.float32)


def _pad_cols(w, nsplit):
    parts = []
    for i in range(nsplit):
        blk = w[:, i * D:(i + 1) * D].reshape(w.shape[0], H, DH)
        blk = jnp.pad(blk, ((0, 0), (0, 0), (0, DHP - DH)))
        parts.append(blk.reshape(w.shape[0], H * DHP))
    return jnp.concatenate(parts, axis=1)


def _pad_rows(w):
    blk = w.reshape(H, DH, D)
    blk = jnp.pad(blk, ((0, 0), (0, DHP - DH), (0, 0)))
    return blk.reshape(H * DHP, D)


def _embed_body(ids_ref, emb_hbm, pe_ref, out_ref, buf, sem):
    i = pl.program_id(0)
    base = i * BT
    copies = []
    for t in range(BT):
        idx = ids_ref[base + t]
        cp = pltpu.make_async_copy(emb_hbm.at[idx], buf.at[t], sem)
        cp.start()
        copies.append(cp)
    for cp in copies:
        cp.wait()
    out_ref[...] = buf[...] * EMB_SCALE + pe_ref[...]


def _embed(ids_flat, emb):
    n = ids_flat.shape[0]
    pe_blocks = S // BT
    return pl.pallas_call(
        _embed_body,
        out_shape=jax.ShapeDtypeStruct((n, D), jnp.float32),
        grid_spec=pltpu.PrefetchScalarGridSpec(
            num_scalar_prefetch=1,
            grid=(n // BT,),
            in_specs=[
                pl.BlockSpec(memory_space=pl.ANY),
                pl.BlockSpec((BT, D), lambda i, ids: (i % pe_blocks, 0)),
            ],
            out_specs=pl.BlockSpec((BT, D), lambda i, ids: (i, 0)),
            scratch_shapes=[
                pltpu.VMEM((BT, D), jnp.float32),
                pltpu.SemaphoreType.DMA,
            ],
        ),
        compiler_params=pltpu.CompilerParams(
            dimension_semantics=("arbitrary",),
        ),
        name="embed_gather",
    )(ids_flat, emb, jnp.asarray(_PE))


def _attn_math(q_ref, q_off, kv_ref, k_off, v_off, padf, causal, o_scr):
    scale = DH ** -0.5
    for h in range(H):
        k = kv_ref[:, k_off + h * DH:k_off + (h + 1) * DH]
        v = kv_ref[:, v_off + h * DH:v_off + (h + 1) * DH]
        for r0 in range(0, S, QC):
            q = q_ref[r0:r0 + QC, q_off + h * DH:q_off + (h + 1) * DH] * scale
            sc = jax.lax.dot_general(q, k, (((1,), (1,)), ((), ())),
                                     preferred_element_type=jnp.float32)
            if causal:
                rows = jax.lax.broadcasted_iota(jnp.int32, (QC, S), 0) + r0
                cols = jax.lax.broadcasted_iota(jnp.int32, (QC, S), 1)
                sc = jnp.where(cols > rows, NEG, sc)
            else:
                sc = sc + padf
            m = jnp.max(sc, axis=-1, keepdims=True)
            p = jnp.exp(sc - m)
            l = jnp.sum(p, axis=-1, keepdims=True)
            o_scr[r0:r0 + QC, h * DH:(h + 1) * DH] = _dot(p, v) / l


def _ffn_tail(state_scr, w1_ref, fb1, w2_ref, fb2, lns, lnb, out_ref, h_scr):
    ft = h_scr.shape[0]
    for r0 in range(0, S, ft):
        xc = state_scr[r0:r0 + ft, :]
        h_scr[...] = jnp.maximum(_dot(xc, w1_ref[...]) + fb1, 0.0)
        y = _dot(h_scr[...], w2_ref[...]) + fb2 + xc
        out_ref[0, r0:r0 + ft, :] = _ln(y, lns, lnb)


def _self_attn_body(x_ref, wqkv_ref, wo_ref, lns_ref, lnb_ref, padf_ref,
                    w1_ref, fb1_ref, w2_ref, fb2_ref, l2s_ref, l2b_ref,
                    out_ref, qkv_scr, o_scr, h_scr, *, causal, with_ffn):
    qkv_scr[...] = _dot(x_ref[0], wqkv_ref[...])
    _attn_math(qkv_scr, 0, qkv_scr, D, 2 * D, padf_ref[0], causal, o_scr)
    s = lns_ref[...]
    b = lnb_ref[...]
    for r0 in range(0, S, QC):
        proj = _dot(o_scr[r0:r0 + QC, :], wo_ref[...])
        y = x_ref[0, r0:r0 + QC, :] + proj
        x2 = _ln(y, s, b)
        if with_ffn:
            o_scr[r0:r0 + QC, :] = x2
        else:
            out_ref[0, r0:r0 + QC, :] = x2
    if with_ffn:
        _ffn_tail(o_scr, w1_ref, fb1_ref[...], w2_ref, fb2_ref[...],
                  l2s_ref[...], l2b_ref[...], out_ref, h_scr)


def _self_attn(x, wqkv, wo, lns, lnb, padf, causal,
               w1=None, fb1=None, w2=None, fb2=None, l2s=None, l2b=None):
    with_ffn = w1 is not None
    if not with_ffn:
        w1 = jnp.zeros((8, DFF), jnp.float32)
        fb1 = jnp.zeros((1, DFF), jnp.float32)
        w2 = jnp.zeros((8, D), jnp.float32)
        fb2 = jnp.zeros((1, D), jnp.float32)
        l2s = jnp.zeros((1, D), jnp.float32)
        l2b = jnp.zeros((1, D), jnp.float32)
    kd = w1.shape[0]
    return pl.pallas_call(
        functools.partial(_self_attn_body, causal=causal, with_ffn=with_ffn),
        out_shape=jax.ShapeDtypeStruct((B, S, D), jnp.float32),
        grid=(B,),
        in_specs=[
            pl.BlockSpec((1, S, D), lambda b: (b, 0, 0)),
            pl.BlockSpec((D, 3 * D), lambda b: (0, 0)),
            pl.BlockSpec((D, D), lambda b: (0, 0)),
            pl.BlockSpec((1, D), lambda b: (0, 0)),
            pl.BlockSpec((1, D), lambda b: (0, 0)),
            pl.BlockSpec((1, 1, S), lambda b: (b, 0, 0)),
            pl.BlockSpec((kd, DFF), lambda b: (0, 0)),
            pl.BlockSpec((1, DFF), lambda b: (0, 0)),
            pl.BlockSpec((w2.shape[0], D), lambda b: (0, 0)),
            pl.BlockSpec((1, D), lambda b: (0, 0)),
            pl.BlockSpec((1, D), lambda b: (0, 0)),
            pl.BlockSpec((1, D), lambda b: (0, 0)),
        ],
        out_specs=pl.BlockSpec((1, S, D), lambda b: (b, 0, 0)),
        scratch_shapes=[
            pltpu.VMEM((S, 3 * D), jnp.float32),
            pltpu.VMEM((S, D), jnp.float32),
            pltpu.VMEM((FT, DFF), jnp.float32),
        ],
        compiler_params=pltpu.CompilerParams(
            dimension_semantics=("parallel",),
            vmem_limit_bytes=52 * 1024 * 1024,
        ),
        name=("enc_layer" if with_ffn else "self_attn_causal"),
    )(x, wqkv, wo, lns, lnb, padf, w1, fb1, w2, fb2, l2s, l2b)


def _cross_attn_body(y_ref, enc_ref, wq_ref, wkv_ref, woc_ref, lns_ref,
                     lnb_ref, padf_ref, w1_ref, fb1_ref, w2_ref, fb2_ref,
                     l2s_ref, l2b_ref, out_ref, q_scr, kv_scr, o_scr, h_scr):
    q_scr[...] = _dot(y_ref[0], wq_ref[...])
    kv_scr[...] = _dot(enc_ref[0], wkv_ref[...])
    _attn_math(q_scr, 0, kv_scr, 0, D, padf_ref[0], False, o_scr)
    s = lns_ref[...]
    b = lnb_ref[...]
    for r0 in range(0, S, QC):
        proj = _dot(o_scr[r0:r0 + QC, :], woc_ref[...])
        y = y_ref[0, r0:r0 + QC, :] + proj
        o_scr[r0:r0 + QC, :] = _ln(y, s, b)
    _ffn_tail(o_scr, w1_ref, fb1_ref[...], w2_ref, fb2_ref[...],
              l2s_ref[...], l2b_ref[...], out_ref, h_scr)


def _cross_attn(y, enc_out, wq, wkv, woc, lns, lnb, padf,
                w1, fb1, w2, fb2, l2s, l2b):
    return pl.pallas_call(
        _cross_attn_body,
        out_shape=jax.ShapeDtypeStruct((B, S, D), jnp.float32),
        grid=(B,),
        in_specs=[
            pl.BlockSpec((1, S, D), lambda b: (b, 0, 0)),
            pl.BlockSpec((1, S, D), lambda b: (b, 0, 0)),
            pl.BlockSpec((D, D), lambda b: (0, 0)),
            pl.BlockSpec((D, 2 * D), lambda b: (0, 0)),
            pl.BlockSpec((D, D), lambda b: (0, 0)),
            pl.BlockSpec((1, D), lambda b: (0, 0)),
            pl.BlockSpec((1, D), lambda b: (0, 0)),
            pl.BlockSpec((1, 1, S), lambda b: (b, 0, 0)),
            pl.BlockSpec((D, DFF), lambda b: (0, 0)),
            pl.BlockSpec((1, DFF), lambda b: (0, 0)),
            pl.BlockSpec((DFF, D), lambda b: (0, 0)),
            pl.BlockSpec((1, D), lambda b: (0, 0)),
            pl.BlockSpec((1, D), lambda b: (0, 0)),
            pl.BlockSpec((1, D), lambda b: (0, 0)),
        ],
        out_specs=pl.BlockSpec((1, S, D), lambda b: (b, 0, 0)),
        scratch_shapes=[
            pltpu.VMEM((S, D), jnp.float32),
            pltpu.VMEM((S, 2 * D), jnp.float32),
            pltpu.VMEM((S, D), jnp.float32),
            pltpu.VMEM((FT, DFF), jnp.float32),
        ],
        compiler_params=pltpu.CompilerParams(
            dimension_semantics=("parallel",),
            vmem_limit_bytes=52 * 1024 * 1024,
        ),
        name="dec_cross_ffn",
    )(y, enc_out, wq, wkv, woc, lns, lnb, padf, w1, fb1, w2, fb2, l2s, l2b)


def _ffn_body(x_ref, w1_ref, b1_ref, w2_ref, b2_ref, lns_ref, lnb_ref,
              out_ref, h_scr):
    h_scr[...] = jnp.maximum(
        _dot(x_ref[...], w1_ref[...]) + b1_ref[...], 0.0)
    y = _dot(h_scr[...], w2_ref[...]) + b2_ref[...] + x_ref[...]
    out_ref[...] = _ln(y, lns_ref[...], lnb_ref[...])


def _ffn(x2d, w1, b1, w2, b2, lns, lnb):
    n = x2d.shape[0]
    return pl.pallas_call(
        _ffn_body,
        out_shape=jax.ShapeDtypeStruct((n, D), jnp.float32),
        grid=(n // FT,),
        in_specs=[
            pl.BlockSpec((FT, D), lambda i: (i, 0)),
            pl.BlockSpec((D, DFF), lambda i: (0, 0)),
            pl.BlockSpec((1, DFF), lambda i: (0, 0)),
            pl.BlockSpec((DFF, D), lambda i: (0, 0)),
            pl.BlockSpec((1, D), lambda i: (0, 0)),
            pl.BlockSpec((1, D), lambda i: (0, 0)),
            pl.BlockSpec((1, D), lambda i: (0, 0)),
        ],
        out_specs=pl.BlockSpec((FT, D), lambda i: (i, 0)),
        scratch_shapes=[pltpu.VMEM((FT, DFF), jnp.float32)],
        compiler_params=pltpu.CompilerParams(
            dimension_semantics=("parallel",),
            vmem_limit_bytes=48 * 1024 * 1024,
        ),
        name="ffn_block",
    )(x2d, w1, b1, w2, b2, lns, lnb)


def _logits_body(x_ref, w_ref, b_ref, out_ref):
    out_ref[...] = (jnp.dot(x_ref[...], w_ref[...],
                            preferred_element_type=jnp.float32)
                    + b_ref[...])


def _logits(x2d, fc_w, fc_b):
    n = x2d.shape[0]
    mt = n // 2
    return pl.pallas_call(
        _logits_body,
        out_shape=jax.ShapeDtypeStruct((n, V), jnp.float32),
        grid=(V // NT, 2),
        in_specs=[
            pl.BlockSpec((mt, D), lambda j, i: (i, 0)),
            pl.BlockSpec((D, NT), lambda j, i: (0, j)),
            pl.BlockSpec((1, NT), lambda j, i: (0, j)),
        ],
        out_specs=pl.BlockSpec((mt, NT), lambda j, i: (i, j)),
        compiler_params=pltpu.CompilerParams(
            dimension_semantics=("parallel", "arbitrary"),
            vmem_limit_bytes=56 * 1024 * 1024,
        ),
        name="vocab_proj",
    )(x2d, fc_w, fc_b.reshape(1, V))


def kernel(batch_src, trg_teacher, src_emb, trg_emb, fc_w, fc_b,
           enc_wqkv, enc_wo, enc_ln1s, enc_ln1b, enc_w1, enc_b1, enc_w2,
           enc_b2, enc_ln2s, enc_ln2b,
           dec_wqkv, dec_wo, dec_ln1s, dec_ln1b, dec_wq, dec_wkv, dec_woc,
           dec_ln2s, dec_ln2b, dec_w1, dec_b1, dec_w2, dec_b2, dec_ln3s,
           dec_ln3b):
    padf = ((batch_src == PAD_ID).astype(jnp.float32) * NEG).reshape(B, 1, S)
    zeros_pad = jnp.zeros_like(padf)

    x = _embed(batch_src.reshape(-1), src_emb).reshape(B, S, D)
    for l in range(L):
        x = _self_attn(x, enc_wqkv[l], enc_wo[l],
                       enc_ln1s[l].reshape(1, D), enc_ln1b[l].reshape(1, D),
                       padf, False,
                       enc_w1[l], enc_b1[l].reshape(1, DFF), enc_w2[l],
                       enc_b2[l].reshape(1, D), enc_ln2s[l].reshape(1, D),
                       enc_ln2b[l].reshape(1, D))
    enc_out = x

    y = _embed(trg_teacher.reshape(-1), trg_emb).reshape(B, S, D)
    for l in range(L):
        y = _self_attn(y, dec_wqkv[l], dec_wo[l],
                       dec_ln1s[l].reshape(1, D), dec_ln1b[l].reshape(1, D),
                       zeros_pad, causal=True)
        y = _cross_attn(y, enc_out, dec_wq[l], dec_wkv[l], dec_woc[l],
                        dec_ln2s[l].reshape(1, D), dec_ln2b[l].reshape(1, D),
                        padf,
                        dec_w1[l], dec_b1[l].reshape(1, DFF), dec_w2[l],
                        dec_b2[l].reshape(1, D), dec_ln3s[l].reshape(1, D),
                        dec_ln3b[l].reshape(1, D))

    return _logits(y.reshape(B * S, D), fc_w, fc_b).reshape(B, S, V)


# speedup vs baseline: 1.5058x; 1.1155x over previous
import functools
import math

import jax
import jax.numpy as jnp
import numpy as np
from jax.experimental import pallas as pl
from jax.experimental.pallas import tpu as pltpu

D = 512
H = 8
L = 6
DFF = 2048
V = 32000
B = 2
S = 1024
DH = D // H
PAD_ID = 0
EMB_SCALE = math.sqrt(D)
NEG = -1e9

QC = 256
NCH = S // QC
BT = 256
NT = 3200
FT = 512


def _posenc(s, d):
    pos = np.arange(s)[:, None].astype(np.float32)
    i = np.arange(0, d, 2)[None, :].astype(np.float32)
    ang = pos / (10000.0 ** (i / d))
    pe = np.zeros((s, d), np.float32)
    pe[:, 0::2] = np.sin(ang)
    pe[:, 1::2] = np.cos(ang)
    return pe


_PE = _posenc(S, D)


def _ln(y, s, b):
    mu = jnp.mean(y, axis=-1, keepdims=True)
    d = y - mu
    var = jnp.mean(d * d, axis=-1, keepdims=True)
    return d * jax.lax.rsqrt(var + 1e-5) * s + b


def _dot(a, w):
    return jnp.dot(a, w, preferred_element_type=jnp.float32)


def _pad_cols(w, nsplit):
    parts = []
    for i in range(nsplit):
        blk = w[:, i * D:(i + 1) * D].reshape(w.shape[0], H, DH)
        blk = jnp.pad(blk, ((0, 0), (0, 0), (0, DHP - DH)))
        parts.append(blk.reshape(w.shape[0], H * DHP))
    return jnp.concatenate(parts, axis=1)


def _pad_rows(w):
    blk = w.reshape(H, DH, D)
    blk = jnp.pad(blk, ((0, 0), (0, DHP - DH), (0, 0)))
    return blk.reshape(H * DHP, D)


def _embed_body(ids_ref, emb_hbm, pe_ref, out_ref, buf, sem):
    i = pl.program_id(0)
    base = i * BT
    copies = []
    for t in range(BT):
        idx = ids_ref[base + t]
        cp = pltpu.make_async_copy(emb_hbm.at[idx], buf.at[t], sem)
        cp.start()
        copies.append(cp)
    for cp in copies:
        cp.wait()
    out_ref[...] = buf[...] * EMB_SCALE + pe_ref[...]


def _embed(ids_flat, emb):
    n = ids_flat.shape[0]
    pe_blocks = S // BT
    return pl.pallas_call(
        _embed_body,
        out_shape=jax.ShapeDtypeStruct((n, D), jnp.float32),
        grid_spec=pltpu.PrefetchScalarGridSpec(
            num_scalar_prefetch=1,
            grid=(n // BT,),
            in_specs=[
                pl.BlockSpec(memory_space=pl.ANY),
                pl.BlockSpec((BT, D), lambda i, ids: (i % pe_blocks, 0)),
            ],
            out_specs=pl.BlockSpec((BT, D), lambda i, ids: (i, 0)),
            scratch_shapes=[
                pltpu.VMEM((BT, D), jnp.float32),
                pltpu.SemaphoreType.DMA,
            ],
        ),
        compiler_params=pltpu.CompilerParams(
            dimension_semantics=("arbitrary",),
        ),
        name="embed_gather",
    )(ids_flat, emb, jnp.asarray(_PE))


def _attn_math(q_ref, q_off, kv_ref, k_off, v_off, padf, causal, o_scr):
    scale = DH ** -0.5
    for h in range(H):
        k = kv_ref[:, k_off + h * DH:k_off + (h + 1) * DH]
        v = kv_ref[:, v_off + h * DH:v_off + (h + 1) * DH]
        for r0 in range(0, S, QC):
            q = q_ref[r0:r0 + QC, q_off + h * DH:q_off + (h + 1) * DH] * scale
            sc = jax.lax.dot_general(q, k, (((1,), (1,)), ((), ())),
                                     preferred_element_type=jnp.float32)
            if causal:
                rows = jax.lax.broadcasted_iota(jnp.int32, (QC, S), 0) + r0
                cols = jax.lax.broadcasted_iota(jnp.int32, (QC, S), 1)
                sc = jnp.where(cols > rows, NEG, sc)
            else:
                sc = sc + padf
            p = jnp.exp(sc)
            l = jnp.sum(p, axis=-1, keepdims=True)
            o_scr[r0:r0 + QC, h * DH:(h + 1) * DH] = _dot(p, v) / l


def _ffn_tail(state_scr, w1_ref, fb1, w2_ref, fb2, lns, lnb, out_ref, h_scr):
    ft = h_scr.shape[0]
    for r0 in range(0, S, ft):
        xc = state_scr[r0:r0 + ft, :]
        h_scr[...] = jnp.maximum(_dot(xc, w1_ref[...]) + fb1, 0.0)
        y = _dot(h_scr[...], w2_ref[...]) + fb2 + xc
        out_ref[0, r0:r0 + ft, :] = _ln(y, lns, lnb)


def _self_attn_body(x_ref, wqkv_ref, wo_ref, lns_ref, lnb_ref, padf_ref,
                    w1_ref, fb1_ref, w2_ref, fb2_ref, l2s_ref, l2b_ref,
                    out_ref, qkv_scr, o_scr, h_scr, *, causal, with_ffn):
    qkv_scr[...] = _dot(x_ref[0], wqkv_ref[...])
    _attn_math(qkv_scr, 0, qkv_scr, D, 2 * D, padf_ref[0], causal, o_scr)
    s = lns_ref[...]
    b = lnb_ref[...]
    for r0 in range(0, S, QC):
        proj = _dot(o_scr[r0:r0 + QC, :], wo_ref[...])
        y = x_ref[0, r0:r0 + QC, :] + proj
        x2 = _ln(y, s, b)
        if with_ffn:
            o_scr[r0:r0 + QC, :] = x2
        else:
            out_ref[0, r0:r0 + QC, :] = x2
    if with_ffn:
        _ffn_tail(o_scr, w1_ref, fb1_ref[...], w2_ref, fb2_ref[...],
                  l2s_ref[...], l2b_ref[...], out_ref, h_scr)


def _self_attn(x, wqkv, wo, lns, lnb, padf, causal,
               w1=None, fb1=None, w2=None, fb2=None, l2s=None, l2b=None):
    with_ffn = w1 is not None
    if not with_ffn:
        w1 = jnp.zeros((8, DFF), jnp.float32)
        fb1 = jnp.zeros((1, DFF), jnp.float32)
        w2 = jnp.zeros((8, D), jnp.float32)
        fb2 = jnp.zeros((1, D), jnp.float32)
        l2s = jnp.zeros((1, D), jnp.float32)
        l2b = jnp.zeros((1, D), jnp.float32)
    kd = w1.shape[0]
    return pl.pallas_call(
        functools.partial(_self_attn_body, causal=causal, with_ffn=with_ffn),
        out_shape=jax.ShapeDtypeStruct((B, S, D), jnp.float32),
        grid=(B,),
        in_specs=[
            pl.BlockSpec((1, S, D), lambda b: (b, 0, 0)),
            pl.BlockSpec((D, 3 * D), lambda b: (0, 0)),
            pl.BlockSpec((D, D), lambda b: (0, 0)),
            pl.BlockSpec((1, D), lambda b: (0, 0)),
            pl.BlockSpec((1, D), lambda b: (0, 0)),
            pl.BlockSpec((1, 1, S), lambda b: (b, 0, 0)),
            pl.BlockSpec((kd, DFF), lambda b: (0, 0)),
            pl.BlockSpec((1, DFF), lambda b: (0, 0)),
            pl.BlockSpec((w2.shape[0], D), lambda b: (0, 0)),
            pl.BlockSpec((1, D), lambda b: (0, 0)),
            pl.BlockSpec((1, D), lambda b: (0, 0)),
            pl.BlockSpec((1, D), lambda b: (0, 0)),
        ],
        out_specs=pl.BlockSpec((1, S, D), lambda b: (b, 0, 0)),
        scratch_shapes=[
            pltpu.VMEM((S, 3 * D), jnp.float32),
            pltpu.VMEM((S, D), jnp.float32),
            pltpu.VMEM((FT, DFF), jnp.float32),
        ],
        compiler_params=pltpu.CompilerParams(
            dimension_semantics=("parallel",),
            vmem_limit_bytes=52 * 1024 * 1024,
        ),
        name=("enc_layer" if with_ffn else "self_attn_causal"),
    )(x, wqkv, wo, lns, lnb, padf, w1, fb1, w2, fb2, l2s, l2b)


def _cross_attn_body(y_ref, enc_ref, wq_ref, wkv_ref, woc_ref, lns_ref,
                     lnb_ref, padf_ref, w1_ref, fb1_ref, w2_ref, fb2_ref,
                     l2s_ref, l2b_ref, out_ref, q_scr, kv_scr, o_scr, h_scr):
    q_scr[...] = _dot(y_ref[0], wq_ref[...])
    kv_scr[...] = _dot(enc_ref[0], wkv_ref[...])
    _attn_math(q_scr, 0, kv_scr, 0, D, padf_ref[0], False, o_scr)
    s = lns_ref[...]
    b = lnb_ref[...]
    for r0 in range(0, S, QC):
        proj = _dot(o_scr[r0:r0 + QC, :], woc_ref[...])
        y = y_ref[0, r0:r0 + QC, :] + proj
        o_scr[r0:r0 + QC, :] = _ln(y, s, b)
    _ffn_tail(o_scr, w1_ref, fb1_ref[...], w2_ref, fb2_ref[...],
              l2s_ref[...], l2b_ref[...], out_ref, h_scr)


def _cross_attn(y, enc_out, wq, wkv, woc, lns, lnb, padf,
                w1, fb1, w2, fb2, l2s, l2b):
    return pl.pallas_call(
        _cross_attn_body,
        out_shape=jax.ShapeDtypeStruct((B, S, D), jnp.float32),
        grid=(B,),
        in_specs=[
            pl.BlockSpec((1, S, D), lambda b: (b, 0, 0)),
            pl.BlockSpec((1, S, D), lambda b: (b, 0, 0)),
            pl.BlockSpec((D, D), lambda b: (0, 0)),
            pl.BlockSpec((D, 2 * D), lambda b: (0, 0)),
            pl.BlockSpec((D, D), lambda b: (0, 0)),
            pl.BlockSpec((1, D), lambda b: (0, 0)),
            pl.BlockSpec((1, D), lambda b: (0, 0)),
            pl.BlockSpec((1, 1, S), lambda b: (b, 0, 0)),
            pl.BlockSpec((D, DFF), lambda b: (0, 0)),
            pl.BlockSpec((1, DFF), lambda b: (0, 0)),
            pl.BlockSpec((DFF, D), lambda b: (0, 0)),
            pl.BlockSpec((1, D), lambda b: (0, 0)),
            pl.BlockSpec((1, D), lambda b: (0, 0)),
            pl.BlockSpec((1, D), lambda b: (0, 0)),
        ],
        out_specs=pl.BlockSpec((1, S, D), lambda b: (b, 0, 0)),
        scratch_shapes=[
            pltpu.VMEM((S, D), jnp.float32),
            pltpu.VMEM((S, 2 * D), jnp.float32),
            pltpu.VMEM((S, D), jnp.float32),
            pltpu.VMEM((FT, DFF), jnp.float32),
        ],
        compiler_params=pltpu.CompilerParams(
            dimension_semantics=("parallel",),
            vmem_limit_bytes=52 * 1024 * 1024,
        ),
        name="dec_cross_ffn",
    )(y, enc_out, wq, wkv, woc, lns, lnb, padf, w1, fb1, w2, fb2, l2s, l2b)


def _ffn_body(x_ref, w1_ref, b1_ref, w2_ref, b2_ref, lns_ref, lnb_ref,
              out_ref, h_scr):
    h_scr[...] = jnp.maximum(
        _dot(x_ref[...], w1_ref[...]) + b1_ref[...], 0.0)
    y = _dot(h_scr[...], w2_ref[...]) + b2_ref[...] + x_ref[...]
    out_ref[...] = _ln(y, lns_ref[...], lnb_ref[...])


def _ffn(x2d, w1, b1, w2, b2, lns, lnb):
    n = x2d.shape[0]
    return pl.pallas_call(
        _ffn_body,
        out_shape=jax.ShapeDtypeStruct((n, D), jnp.float32),
        grid=(n // FT,),
        in_specs=[
            pl.BlockSpec((FT, D), lambda i: (i, 0)),
            pl.BlockSpec((D, DFF), lambda i: (0, 0)),
            pl.BlockSpec((1, DFF), lambda i: (0, 0)),
            pl.BlockSpec((DFF, D), lambda i: (0, 0)),
            pl.BlockSpec((1, D), lambda i: (0, 0)),
            pl.BlockSpec((1, D), lambda i: (0, 0)),
            pl.BlockSpec((1, D), lambda i: (0, 0)),
        ],
        out_specs=pl.BlockSpec((FT, D), lambda i: (i, 0)),
        scratch_shapes=[pltpu.VMEM((FT, DFF), jnp.float32)],
        compiler_params=pltpu.CompilerParams(
            dimension_semantics=("parallel",),
            vmem_limit_bytes=48 * 1024 * 1024,
        ),
        name="ffn_block",
    )(x2d, w1, b1, w2, b2, lns, lnb)


def _logits_body(x_ref, w_ref, b_ref, out_ref):
    out_ref[...] = (jnp.dot(x_ref[...], w_ref[...],
                            preferred_element_type=jnp.float32)
                    + b_ref[...])


def _logits(x2d, fc_w, fc_b):
    n = x2d.shape[0]
    mt = n // 2
    return pl.pallas_call(
        _logits_body,
        out_shape=jax.ShapeDtypeStruct((n, V), jnp.float32),
        grid=(V // NT, 2),
        in_specs=[
            pl.BlockSpec((mt, D), lambda j, i: (i, 0)),
            pl.BlockSpec((D, NT), lambda j, i: (0, j)),
            pl.BlockSpec((1, NT), lambda j, i: (0, j)),
        ],
        out_specs=pl.BlockSpec((mt, NT), lambda j, i: (i, j)),
        compiler_params=pltpu.CompilerParams(
            dimension_semantics=("parallel", "arbitrary"),
            vmem_limit_bytes=56 * 1024 * 1024,
        ),
        name="vocab_proj",
    )(x2d, fc_w, fc_b.reshape(1, V))


def kernel(batch_src, trg_teacher, src_emb, trg_emb, fc_w, fc_b,
           enc_wqkv, enc_wo, enc_ln1s, enc_ln1b, enc_w1, enc_b1, enc_w2,
           enc_b2, enc_ln2s, enc_ln2b,
           dec_wqkv, dec_wo, dec_ln1s, dec_ln1b, dec_wq, dec_wkv, dec_woc,
           dec_ln2s, dec_ln2b, dec_w1, dec_b1, dec_w2, dec_b2, dec_ln3s,
           dec_ln3b):
    padf = ((batch_src == PAD_ID).astype(jnp.float32) * NEG).reshape(B, 1, S)
    zeros_pad = jnp.zeros_like(padf)

    x = _embed(batch_src.reshape(-1), src_emb).reshape(B, S, D)
    for l in range(L):
        x = _self_attn(x, enc_wqkv[l], enc_wo[l],
                       enc_ln1s[l].reshape(1, D), enc_ln1b[l].reshape(1, D),
                       padf, False,
                       enc_w1[l], enc_b1[l].reshape(1, DFF), enc_w2[l],
                       enc_b2[l].reshape(1, D), enc_ln2s[l].reshape(1, D),
                       enc_ln2b[l].reshape(1, D))
    enc_out = x

    y = _embed(trg_teacher.reshape(-1), trg_emb).reshape(B, S, D)
    for l in range(L):
        y = _self_attn(y, dec_wqkv[l], dec_wo[l],
                       dec_ln1s[l].reshape(1, D), dec_ln1b[l].reshape(1, D),
                       zeros_pad, causal=True)
        y = _cross_attn(y, enc_out, dec_wq[l], dec_wkv[l], dec_woc[l],
                        dec_ln2s[l].reshape(1, D), dec_ln2b[l].reshape(1, D),
                        padf,
                        dec_w1[l], dec_b1[l].reshape(1, DFF), dec_w2[l],
                        dec_b2[l].reshape(1, D), dec_ln3s[l].reshape(1, D),
                        dec_ln3b[l].reshape(1, D))

    return _logits(y.reshape(B * S, D), fc_w, fc_b).reshape(B, S, V)


# speedup vs baseline: 1.5060x; 1.0002x over previous
import functools
import math

import jax
import jax.numpy as jnp
import numpy as np
from jax.experimental import pallas as pl
from jax.experimental.pallas import tpu as pltpu

D = 512
H = 8
L = 6
DFF = 2048
V = 32000
B = 2
S = 1024
DH = D // H
PAD_ID = 0
EMB_SCALE = math.sqrt(D)
NEG = -1e9

QC = 256
BT = 256
NT = 3200
FT = 512


def _posenc(s, d):
    pos = np.arange(s)[:, None].astype(np.float32)
    i = np.arange(0, d, 2)[None, :].astype(np.float32)
    ang = pos / (10000.0 ** (i / d))
    pe = np.zeros((s, d), np.float32)
    pe[:, 0::2] = np.sin(ang)
    pe[:, 1::2] = np.cos(ang)
    return pe


_PE = _posenc(S, D)


def _ln(y, s, b):
    mu = jnp.mean(y, axis=-1, keepdims=True)
    d = y - mu
    var = jnp.mean(d * d, axis=-1, keepdims=True)
    return d * jax.lax.rsqrt(var + 1e-5) * s + b


def _dot(a, w):
    return jnp.dot(a, w, preferred_element_type=jnp.float32)


def _embed_body(ids_ref, emb_hbm, pe_ref, out_ref, buf, sem):
    i = pl.program_id(0)
    base = i * BT
    copies = []
    for t in range(BT):
        idx = ids_ref[base + t]
        cp = pltpu.make_async_copy(emb_hbm.at[idx], buf.at[t], sem)
        cp.start()
        copies.append(cp)
    for cp in copies:
        cp.wait()
    out_ref[...] = buf[...] * EMB_SCALE + pe_ref[...]


def _embed(ids_flat, emb):
    n = ids_flat.shape[0]
    pe_blocks = S // BT
    return pl.pallas_call(
        _embed_body,
        out_shape=jax.ShapeDtypeStruct((n, D), jnp.float32),
        grid_spec=pltpu.PrefetchScalarGridSpec(
            num_scalar_prefetch=1,
            grid=(n // BT,),
            in_specs=[
                pl.BlockSpec(memory_space=pl.ANY),
                pl.BlockSpec((BT, D), lambda i, ids: (i % pe_blocks, 0)),
            ],
            out_specs=pl.BlockSpec((BT, D), lambda i, ids: (i, 0)),
            scratch_shapes=[
                pltpu.VMEM((BT, D), jnp.float32),
                pltpu.SemaphoreType.DMA,
            ],
        ),
        compiler_params=pltpu.CompilerParams(
            dimension_semantics=("arbitrary",),
        ),
        name="embed_gather",
    )(ids_flat, emb, jnp.asarray(_PE))


def _attn_math(q_ref, q_off, kv_ref, k_off, v_off, padf, causal, o_scr):
    scale = DH ** -0.5
    for h in range(H):
        k = kv_ref[:, k_off + h * DH:k_off + (h + 1) * DH]
        v = kv_ref[:, v_off + h * DH:v_off + (h + 1) * DH]
        for r0 in range(0, S, QC):
            q = q_ref[r0:r0 + QC, q_off + h * DH:q_off + (h + 1) * DH] * scale
            sc = jax.lax.dot_general(q, k, (((1,), (1,)), ((), ())),
                                     preferred_element_type=jnp.float32)
            if causal:
                rows = jax.lax.broadcasted_iota(jnp.int32, (QC, S), 0) + r0
                cols = jax.lax.broadcasted_iota(jnp.int32, (QC, S), 1)
                sc = jnp.where(cols > rows, NEG, sc)
            else:
                sc = sc + padf
            p = jnp.exp(sc)
            l = jnp.sum(p, axis=-1, keepdims=True)
            o_scr[r0:r0 + QC, h * DH:(h + 1) * DH] = _dot(p, v) / l


def _ffn_tail(state_scr, w1_ref, fb1, w2_ref, fb2, lns, lnb, out_ref, h_scr):
    ft = h_scr.shape[0]
    for r0 in range(0, S, ft):
        xc = state_scr[r0:r0 + ft, :]
        h_scr[...] = jnp.maximum(_dot(xc, w1_ref[...]) + fb1, 0.0)
        y = _dot(h_scr[...], w2_ref[...]) + fb2 + xc
        out_ref[0, r0:r0 + ft, :] = _ln(y, lns, lnb)


def _self_attn_body(x_ref, wqkv_ref, wo_ref, lns_ref, lnb_ref, padf_ref,
                    w1_ref, fb1_ref, w2_ref, fb2_ref, l2s_ref, l2b_ref,
                    out_ref, qkv_scr, o_scr, h_scr, *, causal, with_ffn):
    qkv_scr[...] = _dot(x_ref[0], wqkv_ref[...])
    _attn_math(qkv_scr, 0, qkv_scr, D, 2 * D, padf_ref[0], causal, o_scr)
    s = lns_ref[...]
    b = lnb_ref[...]
    for r0 in range(0, S, QC):
        proj = _dot(o_scr[r0:r0 + QC, :], wo_ref[...])
        y = x_ref[0, r0:r0 + QC, :] + proj
        x2 = _ln(y, s, b)
        if with_ffn:
            o_scr[r0:r0 + QC, :] = x2
        else:
            out_ref[0, r0:r0 + QC, :] = x2
    if with_ffn:
        _ffn_tail(o_scr, w1_ref, fb1_ref[...], w2_ref, fb2_ref[...],
                  l2s_ref[...], l2b_ref[...], out_ref, h_scr)


def _self_attn(x, wqkv, wo, lns, lnb, padf, causal,
               w1=None, fb1=None, w2=None, fb2=None, l2s=None, l2b=None):
    with_ffn = w1 is not None
    if not with_ffn:
        w1 = jnp.zeros((8, DFF), jnp.float32)
        fb1 = jnp.zeros((1, DFF), jnp.float32)
        w2 = jnp.zeros((8, D), jnp.float32)
        fb2 = jnp.zeros((1, D), jnp.float32)
        l2s = jnp.zeros((1, D), jnp.float32)
        l2b = jnp.zeros((1, D), jnp.float32)
    kd = w1.shape[0]
    return pl.pallas_call(
        functools.partial(_self_attn_body, causal=causal, with_ffn=with_ffn),
        out_shape=jax.ShapeDtypeStruct((B, S, D), jnp.float32),
        grid=(B,),
        in_specs=[
            pl.BlockSpec((1, S, D), lambda b: (b, 0, 0)),
            pl.BlockSpec((D, 3 * D), lambda b: (0, 0)),
            pl.BlockSpec((D, D), lambda b: (0, 0)),
            pl.BlockSpec((1, D), lambda b: (0, 0)),
            pl.BlockSpec((1, D), lambda b: (0, 0)),
            pl.BlockSpec((1, 1, S), lambda b: (b, 0, 0)),
            pl.BlockSpec((kd, DFF), lambda b: (0, 0)),
            pl.BlockSpec((1, DFF), lambda b: (0, 0)),
            pl.BlockSpec((w2.shape[0], D), lambda b: (0, 0)),
            pl.BlockSpec((1, D), lambda b: (0, 0)),
            pl.BlockSpec((1, D), lambda b: (0, 0)),
            pl.BlockSpec((1, D), lambda b: (0, 0)),
        ],
        out_specs=pl.BlockSpec((1, S, D), lambda b: (b, 0, 0)),
        scratch_shapes=[
            pltpu.VMEM((S, 3 * D), jnp.float32),
            pltpu.VMEM((S, D), jnp.float32),
            pltpu.VMEM((FT, DFF), jnp.float32),
        ],
        compiler_params=pltpu.CompilerParams(
            dimension_semantics=("parallel",),
            vmem_limit_bytes=52 * 1024 * 1024,
        ),
        name=("enc_layer" if with_ffn else "self_attn_causal"),
    )(x, wqkv, wo, lns, lnb, padf, w1, fb1, w2, fb2, l2s, l2b)


def _cross_attn_body(y_ref, enc_ref, wq_ref, wkv_ref, woc_ref, lns_ref,
                     lnb_ref, padf_ref, w1_ref, fb1_ref, w2_ref, fb2_ref,
                     l2s_ref, l2b_ref, out_ref, q_scr, kv_scr, o_scr, h_scr):
    q_scr[...] = _dot(y_ref[0], wq_ref[...])
    kv_scr[...] = _dot(enc_ref[0], wkv_ref[...])
    _attn_math(q_scr, 0, kv_scr, 0, D, padf_ref[0], False, o_scr)
    s = lns_ref[...]
    b = lnb_ref[...]
    for r0 in range(0, S, QC):
        proj = _dot(o_scr[r0:r0 + QC, :], woc_ref[...])
        y = y_ref[0, r0:r0 + QC, :] + proj
        o_scr[r0:r0 + QC, :] = _ln(y, s, b)
    _ffn_tail(o_scr, w1_ref, fb1_ref[...], w2_ref, fb2_ref[...],
              l2s_ref[...], l2b_ref[...], out_ref, h_scr)


def _cross_attn(y, enc_out, wq, wkv, woc, lns, lnb, padf,
                w1, fb1, w2, fb2, l2s, l2b):
    return pl.pallas_call(
        _cross_attn_body,
        out_shape=jax.ShapeDtypeStruct((B, S, D), jnp.float32),
        grid=(B,),
        in_specs=[
            pl.BlockSpec((1, S, D), lambda b: (b, 0, 0)),
            pl.BlockSpec((1, S, D), lambda b: (b, 0, 0)),
            pl.BlockSpec((D, D), lambda b: (0, 0)),
            pl.BlockSpec((D, 2 * D), lambda b: (0, 0)),
            pl.BlockSpec((D, D), lambda b: (0, 0)),
            pl.BlockSpec((1, D), lambda b: (0, 0)),
            pl.BlockSpec((1, D), lambda b: (0, 0)),
            pl.BlockSpec((1, 1, S), lambda b: (b, 0, 0)),
            pl.BlockSpec((D, DFF), lambda b: (0, 0)),
            pl.BlockSpec((1, DFF), lambda b: (0, 0)),
            pl.BlockSpec((DFF, D), lambda b: (0, 0)),
            pl.BlockSpec((1, D), lambda b: (0, 0)),
            pl.BlockSpec((1, D), lambda b: (0, 0)),
            pl.BlockSpec((1, D), lambda b: (0, 0)),
        ],
        out_specs=pl.BlockSpec((1, S, D), lambda b: (b, 0, 0)),
        scratch_shapes=[
            pltpu.VMEM((S, D), jnp.float32),
            pltpu.VMEM((S, 2 * D), jnp.float32),
            pltpu.VMEM((S, D), jnp.float32),
            pltpu.VMEM((FT, DFF), jnp.float32),
        ],
        compiler_params=pltpu.CompilerParams(
            dimension_semantics=("parallel",),
            vmem_limit_bytes=52 * 1024 * 1024,
        ),
        name="dec_cross_ffn",
    )(y, enc_out, wq, wkv, woc, lns, lnb, padf, w1, fb1, w2, fb2, l2s, l2b)


def _logits_body(x_ref, w_ref, b_ref, out_ref):
    out_ref[...] = (jnp.dot(x_ref[...], w_ref[...],
                            preferred_element_type=jnp.float32)
                    + b_ref[...])


def _logits(x2d, fc_w, fc_b):
    n = x2d.shape[0]
    mt = n // 2
    return pl.pallas_call(
        _logits_body,
        out_shape=jax.ShapeDtypeStruct((n, V), jnp.float32),
        grid=(V // NT, 2),
        in_specs=[
            pl.BlockSpec((mt, D), lambda j, i: (i, 0)),
            pl.BlockSpec((D, NT), lambda j, i: (0, j)),
            pl.BlockSpec((1, NT), lambda j, i: (0, j)),
        ],
        out_specs=pl.BlockSpec((mt, NT), lambda j, i: (i, j)),
        compiler_params=pltpu.CompilerParams(
            dimension_semantics=("parallel", "arbitrary"),
            vmem_limit_bytes=56 * 1024 * 1024,
        ),
        name="vocab_proj",
    )(x2d, fc_w, fc_b.reshape(1, V))


def kernel(batch_src, trg_teacher, src_emb, trg_emb, fc_w, fc_b,
           enc_wqkv, enc_wo, enc_ln1s, enc_ln1b, enc_w1, enc_b1, enc_w2,
           enc_b2, enc_ln2s, enc_ln2b,
           dec_wqkv, dec_wo, dec_ln1s, dec_ln1b, dec_wq, dec_wkv, dec_woc,
           dec_ln2s, dec_ln2b, dec_w1, dec_b1, dec_w2, dec_b2, dec_ln3s,
           dec_ln3b):
    padf = ((batch_src == PAD_ID).astype(jnp.float32) * NEG).reshape(B, 1, S)
    zeros_pad = jnp.zeros_like(padf)

    x = _embed(batch_src.reshape(-1), src_emb).reshape(B, S, D)
    for l in range(L):
        x = _self_attn(x, enc_wqkv[l], enc_wo[l],
                       enc_ln1s[l].reshape(1, D), enc_ln1b[l].reshape(1, D),
                       padf, False,
                       enc_w1[l], enc_b1[l].reshape(1, DFF), enc_w2[l],
                       enc_b2[l].reshape(1, D), enc_ln2s[l].reshape(1, D),
                       enc_ln2b[l].reshape(1, D))
    enc_out = x

    y = _embed(trg_teacher.reshape(-1), trg_emb).reshape(B, S, D)
    for l in range(L):
        y = _self_attn(y, dec_wqkv[l], dec_wo[l],
                       dec_ln1s[l].reshape(1, D), dec_ln1b[l].reshape(1, D),
                       zeros_pad, causal=True)
        y = _cross_attn(y, enc_out, dec_wq[l], dec_wkv[l], dec_woc[l],
                        dec_ln2s[l].reshape(1, D), dec_ln2b[l].reshape(1, D),
                        padf,
                        dec_w1[l], dec_b1[l].reshape(1, DFF), dec_w2[l],
                        dec_b2[l].reshape(1, D), dec_ln3s[l].reshape(1, D),
                        dec_ln3b[l].reshape(1, D))

    return _logits(y.reshape(B * S, D), fc_w, fc_b).reshape(B, S, V)
